# baseline (speedup 1.0000x reference)
import functools

import jax
import jax.numpy as jnp
from jax import lax
from jax.experimental import pallas as pl
from jax.experimental.pallas import tpu as pltpu
from jax.experimental.pallas import tpu_sc as plsc

T = 8192
D = 2048
O = 2048
E = 8
K = 2

B = 512
HB = 1024
H2 = 2 * O
NH = H2 // HB
G = (T * K) // B + E
N_PAD = G * B

NC = 2
NS = 16
NW = NC * NS


def _gelu(v):
  return 0.5 * v * (1.0 + lax.erf(v * (2.0 ** -0.5)))


_BT = 1024


def _gate_body(x_ref, w1_ref, b1_ref, w2_ref, b2_ref,
               i0_ref, i1_ref, w0_ref, w1o_ref):
  x = x_ref[...]
  h = jnp.dot(x, w1_ref[...], preferred_element_type=jnp.float32)
  h = _gelu(h + b1_ref[...])
  logits = jnp.dot(h, w2_ref[...], preferred_element_type=jnp.float32)
  logits = logits + b2_ref[...]
  lane = lax.broadcasted_iota(jnp.int32, logits.shape, 1)
  neg = jnp.float32(-1e30)
  logits = jnp.where(lane < E, logits, neg)
  m0 = jnp.max(logits, axis=1, keepdims=True)
  i0 = jnp.min(jnp.where(logits == m0, lane, E), axis=1)
  l2 = jnp.where(lane == i0[:, None], neg, logits)
  m1 = jnp.max(l2, axis=1, keepdims=True)
  i1 = jnp.min(jnp.where(l2 == m1, lane, E), axis=1)
  w0 = jax.nn.sigmoid(m0[:, 0] - m1[:, 0])
  i0_ref[...] = i0
  i1_ref[...] = i1
  w0_ref[...] = w0
  w1o_ref[...] = 1.0 - w0


def _gate(x, gW1, gb1, gW2, gb2):
  pad1 = 128 - 2 * E
  gW1p = jnp.pad(gW1, ((0, 0), (0, pad1)))
  gb1p = jnp.pad(gb1, (0, pad1)).reshape(1, 128)
  gW2p = jnp.pad(gW2, ((0, pad1), (0, 128 - E)))
  gb2p = jnp.pad(gb2, (0, 128 - E)).reshape(1, 128)
  grid = (T // _BT,)
  out = pl.pallas_call(
      _gate_body,
      grid=grid,
      in_specs=[
          pl.BlockSpec((_BT, D), lambda t: (t, 0)),
          pl.BlockSpec((D, 128), lambda t: (0, 0)),
          pl.BlockSpec((1, 128), lambda t: (0, 0)),
          pl.BlockSpec((128, 128), lambda t: (0, 0)),
          pl.BlockSpec((1, 128), lambda t: (0, 0)),
      ],
      out_specs=[
          pl.BlockSpec((_BT,), lambda t: (t,)),
          pl.BlockSpec((_BT,), lambda t: (t,)),
          pl.BlockSpec((_BT,), lambda t: (t,)),
          pl.BlockSpec((_BT,), lambda t: (t,)),
      ],
      out_shape=[
          jax.ShapeDtypeStruct((T,), jnp.int32),
          jax.ShapeDtypeStruct((T,), jnp.int32),
          jax.ShapeDtypeStruct((T,), jnp.float32),
          jax.ShapeDtypeStruct((T,), jnp.float32),
      ],
  )(x, gW1p, gb1p, gW2p, gb2p)
  return out


def _cast_body(w1_ref, w2_ref, o1_ref, o2_ref):
  o1_ref[...] = w1_ref[...].astype(jnp.bfloat16)
  o2_ref[...] = w2_ref[...].astype(jnp.bfloat16)


def _cast_weights(eW1, eW2):
  return pl.pallas_call(
      _cast_body,
      grid=(E, NH),
      in_specs=[
          pl.BlockSpec((1, D, HB), lambda e, h: (e, 0, h)),
          pl.BlockSpec((1, HB, O), lambda e, h: (e, h, 0)),
      ],
      out_specs=[
          pl.BlockSpec((1, D, HB), lambda e, h: (e, 0, h)),
          pl.BlockSpec((1, HB, O), lambda e, h: (e, h, 0)),
      ],
      out_shape=[
          jax.ShapeDtypeStruct((E, D, H2), jnp.bfloat16),
          jax.ShapeDtypeStruct((E, H2, O), jnp.bfloat16),
      ],
  )(eW1, eW2)


def _routing(i0, i1, w0, w1):
  e_flat = jnp.stack([i0, i1], axis=1).reshape(-1)
  w_flat = jnp.stack([w0, w1], axis=1).reshape(-1)
  tok_flat = jnp.repeat(jnp.arange(T, dtype=jnp.int32), K)
  oh = (e_flat[:, None] == jnp.arange(E, dtype=jnp.int32)[None, :])
  csum = jnp.cumsum(oh.astype(jnp.int32), axis=0)
  rank = jnp.take_along_axis(csum, e_flat[:, None], axis=1)[:, 0] - 1
  counts = csum[-1]
  pc = ((counts + B - 1) // B) * B
  pstart = jnp.concatenate(
      [jnp.zeros((1,), jnp.int32), jnp.cumsum(pc)[:-1].astype(jnp.int32)])
  p = pstart[e_flat] + rank
  tok_pad = jnp.zeros((N_PAD,), jnp.int32).at[p].set(
      tok_flat, unique_indices=True)
  w_pad = jnp.zeros((N_PAD,), jnp.float32).at[p].set(
      w_flat, unique_indices=True)
  tile_base = jnp.arange(G, dtype=jnp.int32) * B
  eot = jnp.sum(tile_base[:, None] >= pstart[None, :], axis=1) - 1
  eot = eot.astype(jnp.int32)
  p2 = p.reshape(T, K)
  return tok_pad, w_pad, eot, p2[:, 0], p2[:, 1]


_DCH = 16
_DROWS = N_PAD // NW
_DNCH = _DROWS // _DCH


@functools.cache
def _sc_mesh():
  return plsc.VectorSubcoreMesh(core_axis_name="c", subcore_axis_name="s")


@functools.cache
def _dispatch_kernel():
  @functools.partial(
      pl.kernel,
      out_type=jax.ShapeDtypeStruct((N_PAD, D), jnp.float32),
      mesh=_sc_mesh(),
      scratch_types=[
          pltpu.VMEM((_DCH,), jnp.int32),
          pltpu.VMEM((_DCH, D), jnp.float32),
          pltpu.SemaphoreType.DMA,
      ],
  )
  def _dispatch(x_hbm, tok_hbm, xs_hbm, idx_v, rows_v, sem):
    wid = lax.axis_index("s") * NC + lax.axis_index("c")
    base = wid * _DROWS

    def body(i, carry):
      off = base + i * _DCH
      pltpu.sync_copy(tok_hbm.at[pl.ds(off, _DCH)], idx_v)
      pltpu.async_copy(x_hbm.at[idx_v], rows_v, sem).wait()
      pltpu.sync_copy(rows_v, xs_hbm.at[pl.ds(off, _DCH)])
      return carry

    lax.fori_loop(0, _DNCH, body, 0)

  return _dispatch


def _moe_body(eot_ref, xs_ref, w1_ref, b1_ref, w2_ref, b2_ref, wp_ref,
              out_ref):
  h = pl.program_id(1)
  x = xs_ref[...].astype(jnp.bfloat16)
  y1 = jnp.dot(x, w1_ref[0], preferred_element_type=jnp.float32)
  y1 = _gelu(y1 + b1_ref[0, 0]).astype(jnp.bfloat16)
  y2 = jnp.dot(y1, w2_ref[0], preferred_element_type=jnp.float32)

  @pl.when(h == 0)
  def _():
    out_ref[...] = y2 + b2_ref[0]

  @pl.when(h > 0)
  def _():
    out_ref[...] = out_ref[...] + y2

  @pl.when(h == NH - 1)
  def _():
    out_ref[...] = out_ref[...] * wp_ref[0, 0][:, None]


def _moe(xs, eW1, eb1, eW2, eb2, w_pad, eot):
  w_pad3 = w_pad.reshape(G, 1, B)
  eb1r = eb1.reshape(E, NH, 1, HB)
  eb2r = eb2.reshape(E, 1, O)
  grid_spec = pltpu.PrefetchScalarGridSpec(
      num_scalar_prefetch=1,
      grid=(G, NH),
      in_specs=[
          pl.BlockSpec((B, D), lambda g, h, eot: (g, 0)),
          pl.BlockSpec((1, D, HB), lambda g, h, eot: (eot[g], 0, h)),
          pl.BlockSpec((1, 1, 1, HB), lambda g, h, eot: (eot[g], h, 0, 0)),
          pl.BlockSpec((1, HB, O), lambda g, h, eot: (eot[g], h, 0)),
          pl.BlockSpec((1, 1, O), lambda g, h, eot: (eot[g], 0, 0)),
          pl.BlockSpec((1, 1, B), lambda g, h, eot: (g, 0, 0)),
      ],
      out_specs=pl.BlockSpec((B, O), lambda g, h, eot: (g, 0)),
  )
  return pl.pallas_call(
      _moe_body,
      grid_spec=grid_spec,
      out_shape=jax.ShapeDtypeStruct((N_PAD, O), jnp.float32),
  )(eot, xs, eW1, eb1r, eW2, eb2r, w_pad3)


_CCH = 16
_CTOK = T // NW
_CNCH = _CTOK // _CCH


@functools.cache
def _combine_kernel():
  @functools.partial(
      pl.kernel,
      out_type=jax.ShapeDtypeStruct((T, O), jnp.float32),
      mesh=_sc_mesh(),
      scratch_types=[
          pltpu.VMEM((_CCH,), jnp.int32),
          pltpu.VMEM((_CCH,), jnp.int32),
          pltpu.VMEM((_CCH, O), jnp.float32),
          pltpu.VMEM((_CCH, O), jnp.float32),
          pltpu.SemaphoreType.DMA,
          pltpu.SemaphoreType.DMA,
      ],
  )
  def _combine(yw_hbm, p0_hbm, p1_hbm, out_hbm,
               i0_v, i1_v, r0_v, r1_v, sem0, sem1):
    wid = lax.axis_index("s") * NC + lax.axis_index("c")
    base = wid * _CTOK

    def body(i, carry):
      off = base + i * _CCH
      pltpu.sync_copy(p0_hbm.at[pl.ds(off, _CCH)], i0_v)
      pltpu.sync_copy(p1_hbm.at[pl.ds(off, _CCH)], i1_v)
      cp0 = pltpu.async_copy(yw_hbm.at[i0_v], r0_v, sem0)
      cp1 = pltpu.async_copy(yw_hbm.at[i1_v], r1_v, sem1)
      cp0.wait()
      cp1.wait()

      def row(ii, c):
        def col(jj, c2):
          s = jj * 64
          for u in range(4):
            sl = pl.ds(s + u * 16, 16)
            r0_v[ii, sl] = r0_v[ii, sl] + r1_v[ii, sl]
          return c2
        return lax.fori_loop(0, O // 64, col, c)

      lax.fori_loop(0, _CCH, row, 0)
      pltpu.sync_copy(r0_v, out_hbm.at[pl.ds(off, _CCH)])
      return carry

    lax.fori_loop(0, _CNCH, body, 0)

  return _combine


def kernel(x, gW1, gb1, gW2, gb2, eW1, eb1, eW2, eb2):
  i0, i1, w0, w1 = _gate(x, gW1, gb1, gW2, gb2)
  tok_pad, w_pad, eot, p0, p1 = _routing(i0, i1, w0, w1)
  xs = _dispatch_kernel()(x, tok_pad)
  yw = _moe(xs, eW1.astype(jnp.bfloat16), eb1,
            eW2.astype(jnp.bfloat16), eb2, w_pad, eot)
  out = _combine_kernel()(yw, p0, p1)
  return out


# speedup vs baseline: 1.1052x; 1.1052x over previous
import functools

import jax
import jax.numpy as jnp
from jax import lax
from jax.experimental import pallas as pl
from jax.experimental.pallas import tpu as pltpu
from jax.experimental.pallas import tpu_sc as plsc

T = 8192
D = 2048
O = 2048
E = 8
K = 2

B = 512
HB = 1024
H2 = 2 * O
NH = H2 // HB
G = (T * K) // B + E
N_PAD = G * B

NC = 2
NS = 16
NW = NC * NS


def _gelu(v):
  return 0.5 * v * (1.0 + lax.erf(v * (2.0 ** -0.5)))


_BT = 1024


def _gate_body(x_ref, w1_ref, b1_ref, w2_ref, b2_ref,
               i0_ref, i1_ref, w0_ref, w1o_ref):
  x = x_ref[...]
  h = jnp.dot(x, w1_ref[...], preferred_element_type=jnp.float32)
  h = _gelu(h + b1_ref[...])
  logits = jnp.dot(h, w2_ref[...], preferred_element_type=jnp.float32)
  logits = logits + b2_ref[...]
  lane = lax.broadcasted_iota(jnp.int32, logits.shape, 1)
  neg = jnp.float32(-1e30)
  logits = jnp.where(lane < E, logits, neg)
  m0 = jnp.max(logits, axis=1, keepdims=True)
  i0 = jnp.min(jnp.where(logits == m0, lane, E), axis=1)
  l2 = jnp.where(lane == i0[:, None], neg, logits)
  m1 = jnp.max(l2, axis=1, keepdims=True)
  i1 = jnp.min(jnp.where(l2 == m1, lane, E), axis=1)
  w0 = jax.nn.sigmoid(m0[:, 0] - m1[:, 0])
  i0_ref[...] = i0
  i1_ref[...] = i1
  w0_ref[...] = w0
  w1o_ref[...] = 1.0 - w0


def _gate(x, gW1, gb1, gW2, gb2):
  pad1 = 128 - 2 * E
  gW1p = jnp.pad(gW1, ((0, 0), (0, pad1)))
  gb1p = jnp.pad(gb1, (0, pad1)).reshape(1, 128)
  gW2p = jnp.pad(gW2, ((0, pad1), (0, 128 - E)))
  gb2p = jnp.pad(gb2, (0, 128 - E)).reshape(1, 128)
  grid = (T // _BT,)
  out = pl.pallas_call(
      _gate_body,
      grid=grid,
      in_specs=[
          pl.BlockSpec((_BT, D), lambda t: (t, 0)),
          pl.BlockSpec((D, 128), lambda t: (0, 0)),
          pl.BlockSpec((1, 128), lambda t: (0, 0)),
          pl.BlockSpec((128, 128), lambda t: (0, 0)),
          pl.BlockSpec((1, 128), lambda t: (0, 0)),
      ],
      out_specs=[
          pl.BlockSpec((_BT,), lambda t: (t,)),
          pl.BlockSpec((_BT,), lambda t: (t,)),
          pl.BlockSpec((_BT,), lambda t: (t,)),
          pl.BlockSpec((_BT,), lambda t: (t,)),
      ],
      out_shape=[
          jax.ShapeDtypeStruct((T,), jnp.int32),
          jax.ShapeDtypeStruct((T,), jnp.int32),
          jax.ShapeDtypeStruct((T,), jnp.float32),
          jax.ShapeDtypeStruct((T,), jnp.float32),
      ],
  )(x, gW1p, gb1p, gW2p, gb2p)
  return out


def _cast_body(w1_ref, w2_ref, o1_ref, o2_ref):
  o1_ref[...] = w1_ref[...].astype(jnp.bfloat16)
  o2_ref[...] = w2_ref[...].astype(jnp.bfloat16)


def _cast_weights(eW1, eW2):
  return pl.pallas_call(
      _cast_body,
      grid=(E, NH),
      in_specs=[
          pl.BlockSpec((1, D, HB), lambda e, h: (e, 0, h)),
          pl.BlockSpec((1, HB, O), lambda e, h: (e, h, 0)),
      ],
      out_specs=[
          pl.BlockSpec((1, D, HB), lambda e, h: (e, 0, h)),
          pl.BlockSpec((1, HB, O), lambda e, h: (e, h, 0)),
      ],
      out_shape=[
          jax.ShapeDtypeStruct((E, D, H2), jnp.bfloat16),
          jax.ShapeDtypeStruct((E, H2, O), jnp.bfloat16),
      ],
  )(eW1, eW2)


def _routing(i0, i1, w0, w1):
  e_flat = jnp.stack([i0, i1], axis=1).reshape(-1)
  w_flat = jnp.stack([w0, w1], axis=1).reshape(-1)
  tok_flat = jnp.repeat(jnp.arange(T, dtype=jnp.int32), K)
  oh = (e_flat[:, None] == jnp.arange(E, dtype=jnp.int32)[None, :])
  csum = jnp.cumsum(oh.astype(jnp.int32), axis=0)
  rank = jnp.take_along_axis(csum, e_flat[:, None], axis=1)[:, 0] - 1
  counts = csum[-1]
  pc = ((counts + B - 1) // B) * B
  pstart = jnp.concatenate(
      [jnp.zeros((1,), jnp.int32), jnp.cumsum(pc)[:-1].astype(jnp.int32)])
  p = pstart[e_flat] + rank
  tok_pad = jnp.zeros((N_PAD,), jnp.int32).at[p].set(
      tok_flat, unique_indices=True)
  w_pad = jnp.zeros((N_PAD,), jnp.float32).at[p].set(
      w_flat, unique_indices=True)
  tile_base = jnp.arange(G, dtype=jnp.int32) * B
  eot = jnp.sum(tile_base[:, None] >= pstart[None, :], axis=1) - 1
  eot = eot.astype(jnp.int32)
  p2 = p.reshape(T, K)
  return tok_pad, w_pad, eot, p2[:, 0], p2[:, 1]


_DCH = 16
_DROWS = N_PAD // NW
_DNCH = _DROWS // _DCH


@functools.cache
def _sc_mesh():
  return plsc.VectorSubcoreMesh(core_axis_name="c", subcore_axis_name="s")


@functools.cache
def _dispatch_kernel():
  @functools.partial(
      pl.kernel,
      out_type=jax.ShapeDtypeStruct((N_PAD, D), jnp.float32),
      mesh=_sc_mesh(),
      scratch_types=[
          pltpu.VMEM((_DCH,), jnp.int32),
          pltpu.VMEM((_DCH, D), jnp.float32),
          pltpu.SemaphoreType.DMA,
      ],
  )
  def _dispatch(x_hbm, tok_hbm, xs_hbm, idx_v, rows_v, sem):
    wid = lax.axis_index("s") * NC + lax.axis_index("c")
    base = wid * _DROWS

    def body(i, carry):
      off = base + i * _DCH
      pltpu.sync_copy(tok_hbm.at[pl.ds(off, _DCH)], idx_v)
      pltpu.async_copy(x_hbm.at[idx_v], rows_v, sem).wait()
      pltpu.sync_copy(rows_v, xs_hbm.at[pl.ds(off, _DCH)])
      return carry

    lax.fori_loop(0, _DNCH, body, 0)

  return _dispatch


def _moe_body(eot_ref, xs_ref, w1_ref, b1_ref, w2_ref, b2_ref, wp_ref,
              out_ref):
  h = pl.program_id(1)
  x = xs_ref[...].astype(jnp.bfloat16)
  y1 = jnp.dot(x, w1_ref[0].astype(jnp.bfloat16),
               preferred_element_type=jnp.float32)
  y1 = _gelu(y1 + b1_ref[0, 0]).astype(jnp.bfloat16)
  y2 = jnp.dot(y1, w2_ref[0].astype(jnp.bfloat16),
               preferred_element_type=jnp.float32)

  @pl.when(h == 0)
  def _():
    out_ref[...] = y2 + b2_ref[0]

  @pl.when(h > 0)
  def _():
    out_ref[...] = out_ref[...] + y2

  @pl.when(h == NH - 1)
  def _():
    out_ref[...] = out_ref[...] * wp_ref[0, 0][:, None]


def _moe(xs, eW1, eb1, eW2, eb2, w_pad, eot):
  w_pad3 = w_pad.reshape(G, 1, B)
  eb1r = eb1.reshape(E, NH, 1, HB)
  eb2r = eb2.reshape(E, 1, O)
  grid_spec = pltpu.PrefetchScalarGridSpec(
      num_scalar_prefetch=1,
      grid=(G, NH),
      in_specs=[
          pl.BlockSpec((B, D), lambda g, h, eot: (g, 0)),
          pl.BlockSpec((1, D, HB), lambda g, h, eot: (eot[g], 0, h)),
          pl.BlockSpec((1, 1, 1, HB), lambda g, h, eot: (eot[g], h, 0, 0)),
          pl.BlockSpec((1, HB, O), lambda g, h, eot: (eot[g], h, 0)),
          pl.BlockSpec((1, 1, O), lambda g, h, eot: (eot[g], 0, 0)),
          pl.BlockSpec((1, 1, B), lambda g, h, eot: (g, 0, 0)),
      ],
      out_specs=pl.BlockSpec((B, O), lambda g, h, eot: (g, 0)),
  )
  return pl.pallas_call(
      _moe_body,
      grid_spec=grid_spec,
      out_shape=jax.ShapeDtypeStruct((N_PAD, O), jnp.float32),
  )(eot, xs, eW1, eb1r, eW2, eb2r, w_pad3)


_CCH = 16
_CTOK = T // NW
_CNCH = _CTOK // _CCH


@functools.cache
def _combine_kernel():
  @functools.partial(
      pl.kernel,
      out_type=jax.ShapeDtypeStruct((T, O), jnp.float32),
      mesh=_sc_mesh(),
      scratch_types=[
          pltpu.VMEM((_CCH,), jnp.int32),
          pltpu.VMEM((_CCH,), jnp.int32),
          pltpu.VMEM((_CCH, O), jnp.float32),
          pltpu.VMEM((_CCH, O), jnp.float32),
          pltpu.SemaphoreType.DMA,
          pltpu.SemaphoreType.DMA,
      ],
  )
  def _combine(yw_hbm, p0_hbm, p1_hbm, out_hbm,
               i0_v, i1_v, r0_v, r1_v, sem0, sem1):
    wid = lax.axis_index("s") * NC + lax.axis_index("c")
    base = wid * _CTOK

    def body(i, carry):
      off = base + i * _CCH
      pltpu.sync_copy(p0_hbm.at[pl.ds(off, _CCH)], i0_v)
      pltpu.sync_copy(p1_hbm.at[pl.ds(off, _CCH)], i1_v)
      cp0 = pltpu.async_copy(yw_hbm.at[i0_v], r0_v, sem0)
      cp1 = pltpu.async_copy(yw_hbm.at[i1_v], r1_v, sem1)
      cp0.wait()
      cp1.wait()

      def row(ii, c):
        def col(jj, c2):
          s = jj * 64
          for u in range(4):
            sl = pl.ds(s + u * 16, 16)
            r0_v[ii, sl] = r0_v[ii, sl] + r1_v[ii, sl]
          return c2
        return lax.fori_loop(0, O // 64, col, c)

      lax.fori_loop(0, _CCH, row, 0)
      pltpu.sync_copy(r0_v, out_hbm.at[pl.ds(off, _CCH)])
      return carry

    lax.fori_loop(0, _CNCH, body, 0)

  return _combine


def kernel(x, gW1, gb1, gW2, gb2, eW1, eb1, eW2, eb2):
  i0, i1, w0, w1 = _gate(x, gW1, gb1, gW2, gb2)
  tok_pad, w_pad, eot, p0, p1 = _routing(i0, i1, w0, w1)
  xs = _dispatch_kernel()(x, tok_pad)
  yw = _moe(xs, eW1, eb1, eW2, eb2, w_pad, eot)
  out = _combine_kernel()(yw, p0, p1)
  return out


# speedup vs baseline: 1.1169x; 1.0106x over previous
import functools

import jax
import jax.numpy as jnp
from jax import lax
from jax.experimental import pallas as pl
from jax.experimental.pallas import tpu as pltpu
from jax.experimental.pallas import tpu_sc as plsc

T = 8192
D = 2048
O = 2048
E = 8
K = 2

B = 512
HB = 1024
H2 = 2 * O
NH = H2 // HB
G = (T * K) // B + E
N_PAD = G * B

NC = 2
NS = 16
NW = NC * NS


def _gelu(v):
  return 0.5 * v * (1.0 + lax.erf(v * (2.0 ** -0.5)))


_BT = 1024


def _gate_body(x_ref, w1_ref, b1_ref, w2_ref, b2_ref,
               i0_ref, i1_ref, w0_ref, w1o_ref):
  x = x_ref[...]
  h = jnp.dot(x, w1_ref[...], preferred_element_type=jnp.float32)
  h = _gelu(h + b1_ref[...])
  logits = jnp.dot(h, w2_ref[...], preferred_element_type=jnp.float32)
  logits = logits + b2_ref[...]
  lane = lax.broadcasted_iota(jnp.int32, logits.shape, 1)
  neg = jnp.float32(-1e30)
  logits = jnp.where(lane < E, logits, neg)
  m0 = jnp.max(logits, axis=1, keepdims=True)
  i0 = jnp.min(jnp.where(logits == m0, lane, E), axis=1)
  l2 = jnp.where(lane == i0[:, None], neg, logits)
  m1 = jnp.max(l2, axis=1, keepdims=True)
  i1 = jnp.min(jnp.where(l2 == m1, lane, E), axis=1)
  w0 = jax.nn.sigmoid(m0[:, 0] - m1[:, 0])
  i0_ref[...] = i0
  i1_ref[...] = i1
  w0_ref[...] = w0
  w1o_ref[...] = 1.0 - w0


def _gate(x, gW1, gb1, gW2, gb2):
  pad1 = 128 - 2 * E
  gW1p = jnp.pad(gW1, ((0, 0), (0, pad1)))
  gb1p = jnp.pad(gb1, (0, pad1)).reshape(1, 128)
  gW2p = jnp.pad(gW2, ((0, pad1), (0, 128 - E)))
  gb2p = jnp.pad(gb2, (0, 128 - E)).reshape(1, 128)
  grid = (T // _BT,)
  out = pl.pallas_call(
      _gate_body,
      grid=grid,
      in_specs=[
          pl.BlockSpec((_BT, D), lambda t: (t, 0)),
          pl.BlockSpec((D, 128), lambda t: (0, 0)),
          pl.BlockSpec((1, 128), lambda t: (0, 0)),
          pl.BlockSpec((128, 128), lambda t: (0, 0)),
          pl.BlockSpec((1, 128), lambda t: (0, 0)),
      ],
      out_specs=[
          pl.BlockSpec((_BT,), lambda t: (t,)),
          pl.BlockSpec((_BT,), lambda t: (t,)),
          pl.BlockSpec((_BT,), lambda t: (t,)),
          pl.BlockSpec((_BT,), lambda t: (t,)),
      ],
      out_shape=[
          jax.ShapeDtypeStruct((T,), jnp.int32),
          jax.ShapeDtypeStruct((T,), jnp.int32),
          jax.ShapeDtypeStruct((T,), jnp.float32),
          jax.ShapeDtypeStruct((T,), jnp.float32),
      ],
  )(x, gW1p, gb1p, gW2p, gb2p)
  return out


def _cast_body(w1_ref, w2_ref, o1_ref, o2_ref):
  o1_ref[...] = w1_ref[...].astype(jnp.bfloat16)
  o2_ref[...] = w2_ref[...].astype(jnp.bfloat16)


def _cast_weights(eW1, eW2):
  return pl.pallas_call(
      _cast_body,
      grid=(E, NH),
      in_specs=[
          pl.BlockSpec((1, D, HB), lambda e, h: (e, 0, h)),
          pl.BlockSpec((1, HB, O), lambda e, h: (e, h, 0)),
      ],
      out_specs=[
          pl.BlockSpec((1, D, HB), lambda e, h: (e, 0, h)),
          pl.BlockSpec((1, HB, O), lambda e, h: (e, h, 0)),
      ],
      out_shape=[
          jax.ShapeDtypeStruct((E, D, H2), jnp.bfloat16),
          jax.ShapeDtypeStruct((E, H2, O), jnp.bfloat16),
      ],
  )(eW1, eW2)


def _routing(i0, i1, w0, w1):
  e_flat = jnp.stack([i0, i1], axis=1).reshape(-1)
  w_flat = jnp.stack([w0, w1], axis=1).reshape(-1)
  tok_flat = jnp.repeat(jnp.arange(T, dtype=jnp.int32), K)
  oh = (e_flat[:, None] == jnp.arange(E, dtype=jnp.int32)[None, :])
  csum = jnp.cumsum(oh.astype(jnp.int32), axis=0)
  rank = jnp.take_along_axis(csum, e_flat[:, None], axis=1)[:, 0] - 1
  counts = csum[-1]
  pc = ((counts + B - 1) // B) * B
  pstart = jnp.concatenate(
      [jnp.zeros((1,), jnp.int32), jnp.cumsum(pc)[:-1].astype(jnp.int32)])
  p = pstart[e_flat] + rank
  tok_pad = jnp.zeros((N_PAD,), jnp.int32).at[p].set(
      tok_flat, unique_indices=True)
  w_pad = jnp.zeros((N_PAD,), jnp.float32).at[p].set(
      w_flat, unique_indices=True)
  tile_base = jnp.arange(G, dtype=jnp.int32) * B
  eot = jnp.sum(tile_base[:, None] >= pstart[None, :], axis=1) - 1
  eot = eot.astype(jnp.int32)
  total_padded = jnp.sum(pc).astype(jnp.int32)
  valid = (tile_base < total_padded).astype(jnp.int32)
  p2 = p.reshape(T, K)
  return tok_pad, w_pad, eot, valid, p2[:, 0], p2[:, 1]


_DCH = 16
_DROWS = N_PAD // NW
_DNCH = _DROWS // _DCH


@functools.cache
def _sc_mesh():
  return plsc.VectorSubcoreMesh(core_axis_name="c", subcore_axis_name="s")


@functools.cache
def _dispatch_kernel():
  @functools.partial(
      pl.kernel,
      out_type=jax.ShapeDtypeStruct((N_PAD, D), jnp.float32),
      mesh=_sc_mesh(),
      scratch_types=[
          pltpu.VMEM((_DCH,), jnp.int32),
          pltpu.VMEM((_DCH,), jnp.int32),
          pltpu.VMEM((_DCH, D), jnp.float32),
          pltpu.VMEM((_DCH, D), jnp.float32),
          pltpu.SemaphoreType.DMA,
          pltpu.SemaphoreType.DMA,
          pltpu.SemaphoreType.DMA,
          pltpu.SemaphoreType.DMA,
      ],
  )
  def _dispatch(x_hbm, tok_hbm, xs_hbm,
                idx0, idx1, buf0, buf1, gs0, gs1, ws0, ws1):
    wid = lax.axis_index("s") * NC + lax.axis_index("c")
    base = wid * _DROWS

    def gather(c, idx, buf, sem):
      off = base + c * _DCH
      pltpu.sync_copy(tok_hbm.at[pl.ds(off, _DCH)], idx)
      pltpu.async_copy(x_hbm.at[idx], buf, sem)

    def gwait(idx, buf, sem):
      pltpu.make_async_copy(x_hbm.at[idx], buf, sem).wait()

    def write(c, buf, sem):
      off = base + c * _DCH
      pltpu.async_copy(buf, xs_hbm.at[pl.ds(off, _DCH)], sem)

    def wwait(c, buf, sem):
      off = base + c * _DCH
      pltpu.make_async_copy(buf, xs_hbm.at[pl.ds(off, _DCH)], sem).wait()

    gather(0, idx0, buf0, gs0)

    def body(k, carry):
      c0 = 2 * k
      c1 = c0 + 1
      gwait(idx0, buf0, gs0)
      gather(c1, idx1, buf1, gs1)
      write(c0, buf0, ws0)
      gwait(idx1, buf1, gs1)
      wwait(c0, buf0, ws0)
      gather(c0 + 2, idx0, buf0, gs0)
      write(c1, buf1, ws1)
      wwait(c1, buf1, ws1)
      return carry

    lax.fori_loop(0, _DNCH // 2 - 1, body, 0)

    c0 = _DNCH - 2
    gwait(idx0, buf0, gs0)
    gather(c0 + 1, idx1, buf1, gs1)
    write(c0, buf0, ws0)
    gwait(idx1, buf1, gs1)
    wwait(c0, buf0, ws0)
    write(c0 + 1, buf1, ws1)
    wwait(c0 + 1, buf1, ws1)

  return _dispatch


def _moe_body(eot_ref, valid_ref, xs_ref, w1_ref, b1_ref, w2_ref, b2_ref,
              wp_ref, out_ref):
  g = pl.program_id(0)
  h = pl.program_id(1)

  @pl.when(valid_ref[g] != 0)
  def _():
    x = xs_ref[...].astype(jnp.bfloat16)
    y1 = jnp.dot(x, w1_ref[0].astype(jnp.bfloat16),
                 preferred_element_type=jnp.float32)
    y1 = _gelu(y1 + b1_ref[0, 0]).astype(jnp.bfloat16)
    y2 = jnp.dot(y1, w2_ref[0].astype(jnp.bfloat16),
                 preferred_element_type=jnp.float32)

    @pl.when(h == 0)
    def _():
      out_ref[...] = y2 + b2_ref[0]

    @pl.when(h > 0)
    def _():
      out_ref[...] = out_ref[...] + y2

    @pl.when(h == NH - 1)
    def _():
      out_ref[...] = out_ref[...] * wp_ref[0, 0][:, None]


def _moe(xs, eW1, eb1, eW2, eb2, w_pad, eot, valid):
  w_pad3 = w_pad.reshape(G, 1, B)
  eb1r = eb1.reshape(E, NH, 1, HB)
  eb2r = eb2.reshape(E, 1, O)
  grid_spec = pltpu.PrefetchScalarGridSpec(
      num_scalar_prefetch=2,
      grid=(G, NH),
      in_specs=[
          pl.BlockSpec((B, D), lambda g, h, eot, v: (g, 0)),
          pl.BlockSpec((1, D, HB), lambda g, h, eot, v: (eot[g], 0, h)),
          pl.BlockSpec((1, 1, 1, HB),
                       lambda g, h, eot, v: (eot[g], h, 0, 0)),
          pl.BlockSpec((1, HB, O), lambda g, h, eot, v: (eot[g], h, 0)),
          pl.BlockSpec((1, 1, O), lambda g, h, eot, v: (eot[g], 0, 0)),
          pl.BlockSpec((1, 1, B), lambda g, h, eot, v: (g, 0, 0)),
      ],
      out_specs=pl.BlockSpec((B, O), lambda g, h, eot, v: (g, 0)),
  )
  return pl.pallas_call(
      _moe_body,
      grid_spec=grid_spec,
      out_shape=jax.ShapeDtypeStruct((N_PAD, O), jnp.float32),
  )(eot, valid, xs, eW1, eb1r, eW2, eb2r, w_pad3)


_CCH = 16
_CTOK = T // NW
_CNCH = _CTOK // _CCH


@functools.cache
def _combine_kernel():
  @functools.partial(
      pl.kernel,
      out_type=jax.ShapeDtypeStruct((T, O), jnp.float32),
      mesh=_sc_mesh(),
      scratch_types=[
          pltpu.VMEM((_CCH,), jnp.int32),
          pltpu.VMEM((_CCH,), jnp.int32),
          pltpu.VMEM((_CCH, O), jnp.float32),
          pltpu.VMEM((_CCH, O), jnp.float32),
          pltpu.SemaphoreType.DMA,
          pltpu.SemaphoreType.DMA,
      ],
  )
  def _combine(yw_hbm, p0_hbm, p1_hbm, out_hbm,
               i0_v, i1_v, r0_v, r1_v, sem0, sem1):
    wid = lax.axis_index("s") * NC + lax.axis_index("c")
    base = wid * _CTOK

    def body(i, carry):
      off = base + i * _CCH
      pltpu.sync_copy(p0_hbm.at[pl.ds(off, _CCH)], i0_v)
      pltpu.sync_copy(p1_hbm.at[pl.ds(off, _CCH)], i1_v)
      cp0 = pltpu.async_copy(yw_hbm.at[i0_v], r0_v, sem0)
      cp1 = pltpu.async_copy(yw_hbm.at[i1_v], r1_v, sem1)
      cp0.wait()
      cp1.wait()

      def row(ii, c):
        def col(jj, c2):
          s = jj * 64
          for u in range(4):
            sl = pl.ds(s + u * 16, 16)
            r0_v[ii, sl] = r0_v[ii, sl] + r1_v[ii, sl]
          return c2
        return lax.fori_loop(0, O // 64, col, c)

      lax.fori_loop(0, _CCH, row, 0)
      pltpu.sync_copy(r0_v, out_hbm.at[pl.ds(off, _CCH)])
      return carry

    lax.fori_loop(0, _CNCH, body, 0)

  return _combine


def kernel(x, gW1, gb1, gW2, gb2, eW1, eb1, eW2, eb2):
  i0, i1, w0, w1 = _gate(x, gW1, gb1, gW2, gb2)
  tok_pad, w_pad, eot, valid, p0, p1 = _routing(i0, i1, w0, w1)
  xs = _dispatch_kernel()(x, tok_pad)
  yw = _moe(xs, eW1, eb1, eW2, eb2, w_pad, eot, valid)
  out = _combine_kernel()(yw, p0, p1)
  return out


# speedup vs baseline: 1.2883x; 1.1535x over previous
import functools

import jax
import jax.numpy as jnp
from jax import lax
from jax.experimental import pallas as pl
from jax.experimental.pallas import tpu as pltpu
from jax.experimental.pallas import tpu_sc as plsc

T = 8192
D = 2048
O = 2048
E = 8
K = 2

B = 512
HB = 1024
H2 = 2 * O
NH = H2 // HB
G = (T * K) // B + E
N_PAD = G * B

NC = 2
NS = 16
NW = NC * NS


def _gelu(v):
  return 0.5 * v * (1.0 + lax.erf(v * (2.0 ** -0.5)))


_BT = 1024


def _gate_body(x_ref, w1_ref, b1_ref, w2_ref, b2_ref,
               i0_ref, i1_ref, w0_ref, w1o_ref, xp_ref):
  x = x_ref[...]
  xb = x.astype(jnp.bfloat16)
  lo = lax.bitcast_convert_type(xb[:, :D // 2], jnp.uint16).astype(jnp.int32)
  hi = lax.bitcast_convert_type(xb[:, D // 2:], jnp.uint16).astype(jnp.int32)
  xp_ref[...] = lo | (hi << 16)
  h = jnp.dot(x, w1_ref[...], preferred_element_type=jnp.float32)
  h = _gelu(h + b1_ref[...])
  logits = jnp.dot(h, w2_ref[...], preferred_element_type=jnp.float32)
  logits = logits + b2_ref[...]
  lane = lax.broadcasted_iota(jnp.int32, logits.shape, 1)
  neg = jnp.float32(-1e30)
  logits = jnp.where(lane < E, logits, neg)
  m0 = jnp.max(logits, axis=1, keepdims=True)
  i0 = jnp.min(jnp.where(logits == m0, lane, E), axis=1)
  l2 = jnp.where(lane == i0[:, None], neg, logits)
  m1 = jnp.max(l2, axis=1, keepdims=True)
  i1 = jnp.min(jnp.where(l2 == m1, lane, E), axis=1)
  w0 = jax.nn.sigmoid(m0[:, 0] - m1[:, 0])
  i0_ref[...] = i0
  i1_ref[...] = i1
  w0_ref[...] = w0
  w1o_ref[...] = 1.0 - w0


def _gate(x, gW1, gb1, gW2, gb2):
  pad1 = 128 - 2 * E
  gW1p = jnp.pad(gW1, ((0, 0), (0, pad1)))
  gb1p = jnp.pad(gb1, (0, pad1)).reshape(1, 128)
  gW2p = jnp.pad(gW2, ((0, pad1), (0, 128 - E)))
  gb2p = jnp.pad(gb2, (0, 128 - E)).reshape(1, 128)
  grid = (T // _BT,)
  out = pl.pallas_call(
      _gate_body,
      grid=grid,
      in_specs=[
          pl.BlockSpec((_BT, D), lambda t: (t, 0)),
          pl.BlockSpec((D, 128), lambda t: (0, 0)),
          pl.BlockSpec((1, 128), lambda t: (0, 0)),
          pl.BlockSpec((128, 128), lambda t: (0, 0)),
          pl.BlockSpec((1, 128), lambda t: (0, 0)),
      ],
      out_specs=[
          pl.BlockSpec((_BT,), lambda t: (t,)),
          pl.BlockSpec((_BT,), lambda t: (t,)),
          pl.BlockSpec((_BT,), lambda t: (t,)),
          pl.BlockSpec((_BT,), lambda t: (t,)),
          pl.BlockSpec((_BT, D // 2), lambda t: (t, 0)),
      ],
      out_shape=[
          jax.ShapeDtypeStruct((T,), jnp.int32),
          jax.ShapeDtypeStruct((T,), jnp.int32),
          jax.ShapeDtypeStruct((T,), jnp.float32),
          jax.ShapeDtypeStruct((T,), jnp.float32),
          jax.ShapeDtypeStruct((T, D // 2), jnp.int32),
      ],
  )(x, gW1p, gb1p, gW2p, gb2p)
  return out


def _cast_body(w1_ref, w2_ref, o1_ref, o2_ref):
  o1_ref[...] = w1_ref[...].astype(jnp.bfloat16)
  o2_ref[...] = w2_ref[...].astype(jnp.bfloat16)


def _cast_weights(eW1, eW2):
  return pl.pallas_call(
      _cast_body,
      grid=(E, NH),
      in_specs=[
          pl.BlockSpec((1, D, HB), lambda e, h: (e, 0, h)),
          pl.BlockSpec((1, HB, O), lambda e, h: (e, h, 0)),
      ],
      out_specs=[
          pl.BlockSpec((1, D, HB), lambda e, h: (e, 0, h)),
          pl.BlockSpec((1, HB, O), lambda e, h: (e, h, 0)),
      ],
      out_shape=[
          jax.ShapeDtypeStruct((E, D, H2), jnp.bfloat16),
          jax.ShapeDtypeStruct((E, H2, O), jnp.bfloat16),
      ],
  )(eW1, eW2)


def _routing(i0, i1, w0, w1):
  e_flat = jnp.stack([i0, i1], axis=1).reshape(-1)
  w_flat = jnp.stack([w0, w1], axis=1).reshape(-1)
  tok_flat = jnp.repeat(jnp.arange(T, dtype=jnp.int32), K)
  oh = (e_flat[:, None] == jnp.arange(E, dtype=jnp.int32)[None, :])
  csum = jnp.cumsum(oh.astype(jnp.int32), axis=0)
  rank = jnp.take_along_axis(csum, e_flat[:, None], axis=1)[:, 0] - 1
  counts = csum[-1]
  pc = ((counts + B - 1) // B) * B
  pstart = jnp.concatenate(
      [jnp.zeros((1,), jnp.int32), jnp.cumsum(pc)[:-1].astype(jnp.int32)])
  p = pstart[e_flat] + rank
  tok_pad = jnp.zeros((N_PAD,), jnp.int32).at[p].set(
      tok_flat, unique_indices=True)
  w_pad = jnp.zeros((N_PAD,), jnp.float32).at[p].set(
      w_flat, unique_indices=True)
  tile_base = jnp.arange(G, dtype=jnp.int32) * B
  eot = jnp.sum(tile_base[:, None] >= pstart[None, :], axis=1) - 1
  eot = eot.astype(jnp.int32)
  total_padded = jnp.sum(pc).astype(jnp.int32)
  valid = (tile_base < total_padded).astype(jnp.int32)
  p2 = p.reshape(T, K)
  return tok_pad, w_pad, eot, valid, p2[:, 0], p2[:, 1]


_DCH = 16
_DROWS = N_PAD // NW
_DNCH = _DROWS // _DCH


@functools.cache
def _sc_mesh():
  return plsc.VectorSubcoreMesh(core_axis_name="c", subcore_axis_name="s")


@functools.cache
def _dispatch_kernel():
  @functools.partial(
      pl.kernel,
      out_type=jax.ShapeDtypeStruct((N_PAD, D // 2), jnp.int32),
      mesh=_sc_mesh(),
      scratch_types=[
          pltpu.VMEM((_DCH,), jnp.int32),
          pltpu.VMEM((_DCH,), jnp.int32),
          pltpu.VMEM((_DCH, D // 2), jnp.int32),
          pltpu.VMEM((_DCH, D // 2), jnp.int32),
          pltpu.SemaphoreType.DMA,
          pltpu.SemaphoreType.DMA,
          pltpu.SemaphoreType.DMA,
          pltpu.SemaphoreType.DMA,
      ],
  )
  def _dispatch(x_hbm, tok_hbm, xs_hbm,
                idx0, idx1, buf0, buf1, gs0, gs1, ws0, ws1):
    wid = lax.axis_index("s") * NC + lax.axis_index("c")
    base = wid * _DROWS

    def gather(c, idx, buf, sem):
      off = base + c * _DCH
      pltpu.sync_copy(tok_hbm.at[pl.ds(off, _DCH)], idx)
      pltpu.async_copy(x_hbm.at[idx], buf, sem)

    def gwait(idx, buf, sem):
      pltpu.make_async_copy(x_hbm.at[idx], buf, sem).wait()

    def write(c, buf, sem):
      off = base + c * _DCH
      pltpu.async_copy(buf, xs_hbm.at[pl.ds(off, _DCH)], sem)

    def wwait(c, buf, sem):
      off = base + c * _DCH
      pltpu.make_async_copy(buf, xs_hbm.at[pl.ds(off, _DCH)], sem).wait()

    gather(0, idx0, buf0, gs0)

    def body(k, carry):
      c0 = 2 * k
      c1 = c0 + 1
      gwait(idx0, buf0, gs0)
      gather(c1, idx1, buf1, gs1)
      write(c0, buf0, ws0)
      gwait(idx1, buf1, gs1)
      wwait(c0, buf0, ws0)
      gather(c0 + 2, idx0, buf0, gs0)
      write(c1, buf1, ws1)
      wwait(c1, buf1, ws1)
      return carry

    lax.fori_loop(0, _DNCH // 2 - 1, body, 0)

    c0 = _DNCH - 2
    gwait(idx0, buf0, gs0)
    gather(c0 + 1, idx1, buf1, gs1)
    write(c0, buf0, ws0)
    gwait(idx1, buf1, gs1)
    wwait(c0, buf0, ws0)
    write(c0 + 1, buf1, ws1)
    wwait(c0 + 1, buf1, ws1)

  return _dispatch


def _moe_body(eot_ref, valid_ref, xs_ref, w1_ref, b1_ref, w2_ref, b2_ref,
              wp_ref, out_ref):
  g = pl.program_id(0)
  h = pl.program_id(1)

  @pl.when(valid_ref[g] != 0)
  def _():
    xw = xs_ref[...]
    xlo = lax.bitcast_convert_type(xw << 16, jnp.float32).astype(jnp.bfloat16)
    xhi = lax.bitcast_convert_type(
        lax.shift_right_logical(xw, 16) << 16, jnp.float32
    ).astype(jnp.bfloat16)
    w1 = w1_ref[0].astype(jnp.bfloat16)
    y1 = (jnp.dot(xlo, w1[:D // 2], preferred_element_type=jnp.float32)
          + jnp.dot(xhi, w1[D // 2:], preferred_element_type=jnp.float32))
    y1 = _gelu(y1 + b1_ref[0, 0]).astype(jnp.bfloat16)
    y2 = jnp.dot(y1, w2_ref[0].astype(jnp.bfloat16),
                 preferred_element_type=jnp.float32)

    @pl.when(h == 0)
    def _():
      out_ref[...] = y2 + b2_ref[0]

    @pl.when(h > 0)
    def _():
      out_ref[...] = out_ref[...] + y2

    @pl.when(h == NH - 1)
    def _():
      out_ref[...] = out_ref[...] * wp_ref[0, 0][:, None]


def _moe(xs, eW1, eb1, eW2, eb2, w_pad, eot, valid):
  w_pad3 = w_pad.reshape(G, 1, B)
  eb1r = eb1.reshape(E, NH, 1, HB)
  eb2r = eb2.reshape(E, 1, O)
  grid_spec = pltpu.PrefetchScalarGridSpec(
      num_scalar_prefetch=2,
      grid=(G, NH),
      in_specs=[
          pl.BlockSpec((B, D // 2), lambda g, h, eot, v: (g, 0)),
          pl.BlockSpec((1, D, HB), lambda g, h, eot, v: (eot[g], 0, h)),
          pl.BlockSpec((1, 1, 1, HB),
                       lambda g, h, eot, v: (eot[g], h, 0, 0)),
          pl.BlockSpec((1, HB, O), lambda g, h, eot, v: (eot[g], h, 0)),
          pl.BlockSpec((1, 1, O), lambda g, h, eot, v: (eot[g], 0, 0)),
          pl.BlockSpec((1, 1, B), lambda g, h, eot, v: (g, 0, 0)),
      ],
      out_specs=pl.BlockSpec((B, O), lambda g, h, eot, v: (g, 0)),
  )
  return pl.pallas_call(
      _moe_body,
      grid_spec=grid_spec,
      out_shape=jax.ShapeDtypeStruct((N_PAD, O), jnp.float32),
  )(eot, valid, xs, eW1, eb1r, eW2, eb2r, w_pad3)


_CCH = 16
_CTOK = T // NW
_CNCH = _CTOK // _CCH


@functools.cache
def _combine_kernel():
  @functools.partial(
      pl.kernel,
      out_type=jax.ShapeDtypeStruct((T, O), jnp.float32),
      mesh=_sc_mesh(),
      scratch_types=[
          pltpu.VMEM((_CCH,), jnp.int32),
          pltpu.VMEM((_CCH,), jnp.int32),
          pltpu.VMEM((_CCH, O), jnp.float32),
          pltpu.VMEM((_CCH, O), jnp.float32),
          pltpu.SemaphoreType.DMA,
          pltpu.SemaphoreType.DMA,
      ],
  )
  def _combine(yw_hbm, p0_hbm, p1_hbm, out_hbm,
               i0_v, i1_v, r0_v, r1_v, sem0, sem1):
    wid = lax.axis_index("s") * NC + lax.axis_index("c")
    base = wid * _CTOK

    def body(i, carry):
      off = base + i * _CCH
      pltpu.sync_copy(p0_hbm.at[pl.ds(off, _CCH)], i0_v)
      pltpu.sync_copy(p1_hbm.at[pl.ds(off, _CCH)], i1_v)
      cp0 = pltpu.async_copy(yw_hbm.at[i0_v], r0_v, sem0)
      cp1 = pltpu.async_copy(yw_hbm.at[i1_v], r1_v, sem1)
      cp0.wait()
      cp1.wait()

      def row(ii, c):
        def col(jj, c2):
          s = jj * 64
          for u in range(4):
            sl = pl.ds(s + u * 16, 16)
            r0_v[ii, sl] = r0_v[ii, sl] + r1_v[ii, sl]
          return c2
        return lax.fori_loop(0, O // 64, col, c)

      lax.fori_loop(0, _CCH, row, 0)
      pltpu.sync_copy(r0_v, out_hbm.at[pl.ds(off, _CCH)])
      return carry

    lax.fori_loop(0, _CNCH, body, 0)

  return _combine


def kernel(x, gW1, gb1, gW2, gb2, eW1, eb1, eW2, eb2):
  i0, i1, w0, w1, xp = _gate(x, gW1, gb1, gW2, gb2)
  tok_pad, w_pad, eot, valid, p0, p1 = _routing(i0, i1, w0, w1)
  xs = _dispatch_kernel()(xp, tok_pad)
  yw = _moe(xs, eW1, eb1, eW2, eb2, w_pad, eot, valid)
  out = _combine_kernel()(yw, p0, p1)
  return out
